# baseline (speedup 1.0000x reference)
import functools

import jax
import jax.numpy as jnp
from jax import lax
from jax.experimental import pallas as pl
from jax.experimental.pallas import tpu as pltpu
from jax.experimental.pallas import tpu_sc as plsc

B = 16384
D = 64
H1 = 128
NC, NS = 2, 16
NW = NC * NS
BPW = B // NW


@functools.cache
def _make_sc_gather():
    mesh = plsc.VectorSubcoreMesh(
        core_axis_name="c", subcore_axis_name="s", num_cores=NC, num_subcores=NS)

    @functools.partial(
        pl.kernel,
        out_type=jax.ShapeDtypeStruct((B, D), jnp.float32),
        mesh=mesh,
        compiler_params=pltpu.CompilerParams(use_tc_tiling_on_sc=False),
        scratch_types=[
            pltpu.VMEM((BPW,), jnp.int32),
            pltpu.VMEM((BPW, D), jnp.float32),
            pltpu.SemaphoreType.DMA,
        ],
    )
    def _sc_gather(idx_hbm, tab_hbm, out_hbm, idx_v, rows_v, sem):
        wid = lax.axis_index("s") * NC + lax.axis_index("c")
        base = wid * BPW
        pltpu.sync_copy(idx_hbm.at[pl.ds(base, BPW)], idx_v)
        copies = [
            pltpu.async_copy(
                tab_hbm.at[idx_v.at[pl.ds(c * 128, 128)]],
                rows_v.at[pl.ds(c * 128, 128)], sem)
            for c in range(BPW // 128)
        ]
        for cp in copies:
            cp.wait()
        pltpu.sync_copy(rows_v, out_hbm.at[pl.ds(base, BPW)])

    return _sc_gather


def _mlp_body(u_ref, i_ref, a1u_ref, a1i_ref, c1_ref, w23_ref, c3_ref, out_ref):
    h = jnp.dot(u_ref[...], a1u_ref[...], preferred_element_type=jnp.float32)
    h = h + jnp.dot(i_ref[...], a1i_ref[...], preferred_element_type=jnp.float32)
    h = jnp.maximum(h + c1_ref[...], 0.0)
    out_ref[...] = (
        jnp.dot(h, w23_ref[...], preferred_element_type=jnp.float32) + c3_ref[...])


def _mlp(u, i, a1u, a1i, c1, w23, c3, bm=2048):
    grid = (B // bm,)
    return pl.pallas_call(
        _mlp_body,
        out_shape=jax.ShapeDtypeStruct((B, 1), jnp.float32),
        grid=grid,
        in_specs=[
            pl.BlockSpec((bm, D), lambda m: (m, 0)),
            pl.BlockSpec((bm, D), lambda m: (m, 0)),
            pl.BlockSpec((D, H1), lambda m: (0, 0)),
            pl.BlockSpec((D, H1), lambda m: (0, 0)),
            pl.BlockSpec((1, H1), lambda m: (0, 0)),
            pl.BlockSpec((H1, 1), lambda m: (0, 0)),
            pl.BlockSpec((1, 1), lambda m: (0, 0)),
        ],
        out_specs=pl.BlockSpec((bm, 1), lambda m: (m, 0)),
    )(u, i, a1u, a1i, c1, w23, c3)


def kernel(user, item, user_table, item_table, g0, be0, W1, b1, g1, be1, W2, b2, W3, b3):
    s = 1.0 / jnp.sqrt(1.0 + 1e-5)
    g0p = g0 * s
    g1p = g1 * s
    A1 = W1 * g0p[None, :] * g1p[:, None]
    c1 = g1p * (W1 @ be0 + b1) + be1
    w23 = (W3 @ W2).T
    c3 = (W3 @ b2 + b3).reshape(1, 1)
    a1u = A1[:, :D].T
    a1i = A1[:, D:].T

    uidx = user.astype(jnp.int32)
    iidx = item.astype(jnp.int32)
    gather = _make_sc_gather()
    i_emb = gather(iidx, item_table)
    u_emb = gather(uidx, user_table)
    out = _mlp(u_emb, i_emb, a1u, a1i, c1.reshape(1, H1), w23, c3)
    return out.reshape(B)


# speedup vs baseline: 1.6378x; 1.6378x over previous
import functools

import jax
import jax.numpy as jnp
from jax import lax
from jax.experimental import pallas as pl
from jax.experimental.pallas import tpu as pltpu
from jax.experimental.pallas import tpu_sc as plsc

B = 16384
D = 64
H1 = 128
NC, NS = 2, 16
NW = NC * NS
BPW = B // NW


@functools.cache
def _make_sc_gather():
    mesh = plsc.VectorSubcoreMesh(
        core_axis_name="c", subcore_axis_name="s", num_cores=NC, num_subcores=NS)

    @functools.partial(
        pl.kernel,
        out_type=jax.ShapeDtypeStruct((B, D), jnp.float32),
        mesh=mesh,
        scratch_types=[
            pltpu.VMEM((BPW,), jnp.int32),
            pltpu.VMEM((BPW // 2, D), jnp.float32),
            pltpu.SemaphoreType.DMA,
        ],
    )
    def _sc_gather(idx_hbm, tab_hbm, out_hbm, idx_v, rows_v, sem):
        wid = lax.axis_index("s") * NC + lax.axis_index("c")
        base = wid * BPW
        pltpu.sync_copy(idx_hbm.at[pl.ds(base, BPW)], idx_v)

        half = BPW // 2
        for h in range(2):
            hb = h * half

            def group(g, _):
                gb = g * 16
                v = idx_v[pl.ds(hb + gb, 16)]
                for k in range(16):
                    pltpu.make_async_copy(
                        tab_hbm.at[pl.ds(v[k], 1)],
                        rows_v.at[pl.ds(gb + k, 1)], sem).start()
                return ()

            lax.fori_loop(0, half // 16, group, ())
            pltpu.make_async_copy(tab_hbm.at[pl.ds(0, half)], rows_v, sem).wait()
            pltpu.sync_copy(rows_v, out_hbm.at[pl.ds(base + hb, half)])

    return _sc_gather


def _mlp_body(u_ref, i_ref, a1u_ref, a1i_ref, c1_ref, w23_ref, c3_ref, out_ref):
    h = jnp.dot(u_ref[...], a1u_ref[...], preferred_element_type=jnp.float32)
    h = h + jnp.dot(i_ref[...], a1i_ref[...], preferred_element_type=jnp.float32)
    h = jnp.maximum(h + c1_ref[...], 0.0)
    out_ref[...] = (
        jnp.dot(h, w23_ref[...], preferred_element_type=jnp.float32) + c3_ref[...])


def _mlp(u, i, a1u, a1i, c1, w23, c3, bm=2048):
    grid = (B // bm,)
    return pl.pallas_call(
        _mlp_body,
        out_shape=jax.ShapeDtypeStruct((B, 1), jnp.float32),
        grid=grid,
        in_specs=[
            pl.BlockSpec((bm, D), lambda m: (m, 0)),
            pl.BlockSpec((bm, D), lambda m: (m, 0)),
            pl.BlockSpec((D, H1), lambda m: (0, 0)),
            pl.BlockSpec((D, H1), lambda m: (0, 0)),
            pl.BlockSpec((1, H1), lambda m: (0, 0)),
            pl.BlockSpec((H1, 1), lambda m: (0, 0)),
            pl.BlockSpec((1, 1), lambda m: (0, 0)),
        ],
        out_specs=pl.BlockSpec((bm, 1), lambda m: (m, 0)),
    )(u, i, a1u, a1i, c1, w23, c3)


def kernel(user, item, user_table, item_table, g0, be0, W1, b1, g1, be1, W2, b2, W3, b3):
    s = 1.0 / jnp.sqrt(1.0 + 1e-5)
    g0p = g0 * s
    g1p = g1 * s
    A1 = W1 * g0p[None, :] * g1p[:, None]
    c1 = g1p * (W1 @ be0 + b1) + be1
    w23 = (W3 @ W2).T
    c3 = (W3 @ b2 + b3).reshape(1, 1)
    a1u = A1[:, :D].T
    a1i = A1[:, D:].T

    uidx = user.astype(jnp.int32)
    iidx = item.astype(jnp.int32)
    gather = _make_sc_gather()
    i_emb = gather(iidx, item_table)
    u_emb = gather(uidx, user_table)
    out = _mlp(u_emb, i_emb, a1u, a1i, c1.reshape(1, H1), w23, c3)
    return out.reshape(B)
